# Initial kernel scaffold; baseline (speedup 1.0000x reference)
import jax
import jax.numpy as jnp
from jax.experimental import pallas as pl


def kernel(x, edge_index, edge_weight, batch, W1, b1, W2, b2, Wm1, bm1, Wm2, bm2):
    raise NotImplementedError("write your pallas kernel here")


# speedup vs baseline: 9.0305x; 9.0305x over previous
import jax
import jax.numpy as jnp
from jax import lax
from jax.experimental import pallas as pl
from jax.experimental.pallas import tpu as pltpu
from jax.experimental.pallas import tpu_sc as plsc

N = 10000
E = 320000
D = 128
H = 64
G = 16
NP = 10240

NUM_CORES = 2
NUM_SUBCORES = 16
NUM_TILES = NUM_CORES * NUM_SUBCORES

EDGES_PER_TILE = E // NUM_TILES
K = 80
NCHUNK = EDGES_PER_TILE // K
ROWS_PER_SUB = NP // NUM_SUBCORES
ZROWS = 32


def _sc_mesh():
    return plsc.VectorSubcoreMesh(core_axis_name="c", subcore_axis_name="s",
                                  num_cores=NUM_CORES, num_subcores=NUM_SUBCORES)


NGRP = NP // 128


def _deg_body(dst_hbm, ew_hbm, out_hbm, dst_v, ew_v, dstrow_v, rows_v, zb_v, deg_sh):
    cid = lax.axis_index("c")
    sid = lax.axis_index("s")

    for r in range(8):
        for j in range(D // 16):
            zb_v[r, pl.ds(j * 16, 16)] = jnp.zeros((16,), jnp.float32)

    @pl.when(sid < NGRP // 8)
    def _():
        pltpu.sync_copy(zb_v, deg_sh.at[pl.ds(sid * 8, 8)])

    plsc.subcore_barrier()

    tile_base = (cid * NUM_SUBCORES + sid) * EDGES_PER_TILE
    iotas = [lax.iota(jnp.int32, 16) + 16 * j for j in range(8)]

    @pl.loop(0, NCHUNK)
    def _chunk(ci):
        base = tile_base + ci * K
        pltpu.sync_copy(dst_hbm.at[pl.ds(base, K)], dst_v)
        pltpu.sync_copy(ew_hbm.at[pl.ds(base, K)], ew_v)

        @pl.loop(0, K // 16)
        def _grp(g):
            sl = pl.ds(g * 16, 16)
            d16 = dst_v[sl]
            dstrow_v[sl] = lax.shift_right_logical(d16, 7)
            ew16 = ew_v[sl]
            lane16 = lax.bitwise_and(d16, 127)
            for e in range(16):
                lane = jnp.broadcast_to(lane16[e:e + 1], (16,))
                w = jnp.broadcast_to(ew16[e:e + 1], (16,))
                row = g * 16 + e
                for j in range(8):
                    rows_v[row, pl.ds(j * 16, 16)] = jnp.where(
                        iotas[j] == lane, w, 0.0)

        pltpu.sync_copy(rows_v, deg_sh.at[dstrow_v], add=True)

    plsc.subcore_barrier()

    @pl.when(sid < NGRP // 8)
    def _():
        pltpu.sync_copy(deg_sh.at[pl.ds(sid * 8, 8)],
                        out_hbm.at[cid, pl.ds(sid * 8, 8)])


def _sc_degree(dst, ew):
    return pl.kernel(
        _deg_body,
        out_type=jax.ShapeDtypeStruct((NUM_CORES, NGRP, 128), jnp.float32),
        mesh=_sc_mesh(),
        scratch_types=[
            pltpu.VMEM((K,), jnp.int32),
            pltpu.VMEM((K,), jnp.float32),
            pltpu.VMEM((K,), jnp.int32),
            pltpu.VMEM((K, 128), jnp.float32),
            pltpu.VMEM((8, 128), jnp.float32),
            pltpu.VMEM_SHARED((NGRP, 128), jnp.float32),
        ],
    )(dst, ew)


def _spmm_body(y_hbm, src_hbm, dst_hbm, ew_hbm, out_hbm,
               src_v, dst_v, ew_v, rows_v, zbuf_v, acc_sh, gsem):
    cid = lax.axis_index("c")
    sid = lax.axis_index("s")

    for r in range(ZROWS):
        for j in range(D // 16):
            zbuf_v[r, pl.ds(j * 16, 16)] = jnp.zeros((16,), jnp.float32)

    row0 = sid * ROWS_PER_SUB

    @pl.loop(0, ROWS_PER_SUB // ZROWS)
    def _zero(i):
        pltpu.sync_copy(zbuf_v, acc_sh.at[pl.ds(row0 + i * ZROWS, ZROWS)])

    plsc.subcore_barrier()

    tile_base = (cid * NUM_SUBCORES + sid) * EDGES_PER_TILE

    @pl.loop(0, NCHUNK)
    def _chunk(ci):
        base = tile_base + ci * K
        pltpu.sync_copy(src_hbm.at[pl.ds(base, K)], src_v)
        pltpu.sync_copy(dst_hbm.at[pl.ds(base, K)], dst_v)
        pltpu.sync_copy(ew_hbm.at[pl.ds(base, K)], ew_v)
        pltpu.async_copy(y_hbm.at[src_v], rows_v, gsem).wait()

        @pl.loop(0, K // 16)
        def _scale(g):
            ew16 = ew_v[pl.ds(g * 16, 16)]
            for e in range(16):
                w = jnp.broadcast_to(ew16[e:e + 1], (16,))
                row = g * 16 + e
                for j in range(D // 16):
                    sl = pl.ds(j * 16, 16)
                    rows_v[row, sl] = rows_v[row, sl] * w

        pltpu.sync_copy(rows_v, acc_sh.at[dst_v], add=True)

    plsc.subcore_barrier()
    pltpu.sync_copy(acc_sh.at[pl.ds(row0, ROWS_PER_SUB)],
                    out_hbm.at[cid, pl.ds(row0, ROWS_PER_SUB)])


def _sc_spmm(y, src, dst, ew):
    return pl.kernel(
        _spmm_body,
        out_type=jax.ShapeDtypeStruct((NUM_CORES, NP, D), jnp.float32),
        mesh=_sc_mesh(),
        scratch_types=[
            pltpu.VMEM((K,), jnp.int32),
            pltpu.VMEM((K,), jnp.int32),
            pltpu.VMEM((K,), jnp.float32),
            pltpu.VMEM((K, D), jnp.float32),
            pltpu.VMEM((ZROWS, D), jnp.float32),
            pltpu.VMEM_SHARED((NP, D), jnp.float32),
            pltpu.SemaphoreType.DMA,
        ],
    )(y, src, dst, ew)


BLK = 1024
NBLK = NP // BLK


def _dinv_from_parts(dp0, dp1):
    deg = dp0 + dp1 + 1.0
    return jnp.where(deg > 0, lax.rsqrt(deg), 0.0)


def _prep_body(dp0_ref, dp1_ref, x_ref, y_ref):
    dinv = _dinv_from_parts(dp0_ref[...], dp1_ref[...])
    y_ref[...] = x_ref[...] * dinv


def _tc_prep(dp0, dp1, x):
    return pl.pallas_call(
        _prep_body,
        grid=(NBLK,),
        in_specs=[
            pl.BlockSpec((BLK, 1), lambda i: (i, 0)),
            pl.BlockSpec((BLK, 1), lambda i: (i, 0)),
            pl.BlockSpec((BLK, D), lambda i: (i, 0)),
        ],
        out_specs=pl.BlockSpec((BLK, D), lambda i: (i, 0)),
        out_shape=jax.ShapeDtypeStruct((NP, D), jnp.float32),
    )(dp0, dp1, x)


def _mid_body(dp0_ref, dp1_ref, p0_ref, p1_ref, y_ref, w_ref, b_ref, o_ref):
    dinv = _dinv_from_parts(dp0_ref[...], dp1_ref[...])
    z = (p0_ref[...] + p1_ref[...] + y_ref[...]) * dinv
    h = jnp.dot(z, w_ref[...], preferred_element_type=jnp.float32,
                precision=lax.Precision.HIGHEST) + b_ref[...]
    o_ref[...] = jnp.maximum(h, 0.0) * dinv


def _tc_mid(dp0, dp1, p0, p1, y, W, b):
    return pl.pallas_call(
        _mid_body,
        grid=(NBLK,),
        in_specs=[
            pl.BlockSpec((BLK, 1), lambda i: (i, 0)),
            pl.BlockSpec((BLK, 1), lambda i: (i, 0)),
            pl.BlockSpec((BLK, D), lambda i: (i, 0)),
            pl.BlockSpec((BLK, D), lambda i: (i, 0)),
            pl.BlockSpec((BLK, D), lambda i: (i, 0)),
            pl.BlockSpec((D, D), lambda i: (0, 0)),
            pl.BlockSpec((1, D), lambda i: (0, 0)),
        ],
        out_specs=pl.BlockSpec((BLK, D), lambda i: (i, 0)),
        out_shape=jax.ShapeDtypeStruct((NP, D), jnp.float32),
    )(dp0, dp1, p0, p1, y, W, b.reshape(1, D))


def _final_body(dp0_ref, dp1_ref, q0_ref, q1_ref, y_ref, w2_ref, b2_ref,
                wm1_ref, bm1_ref, wm2_ref, bm2_ref, batch_ref,
                out_ref, sums_ref, cnt_ref):
    i = pl.program_id(0)
    dinv = _dinv_from_parts(dp0_ref[...], dp1_ref[...])
    z = (q0_ref[...] + q1_ref[...] + y_ref[...]) * dinv
    h2 = jnp.dot(z, w2_ref[...], preferred_element_type=jnp.float32,
                 precision=lax.Precision.HIGHEST) + b2_ref[...]
    t = jnp.maximum(jnp.dot(h2, wm1_ref[...], preferred_element_type=jnp.float32,
                            precision=lax.Precision.HIGHEST) + bm1_ref[...], 0.0)
    o = jnp.dot(t, wm2_ref[...], preferred_element_type=jnp.float32,
                precision=lax.Precision.HIGHEST) + bm2_ref[...]
    b = batch_ref[...][0]
    onehot = (b[0][:, None] == lax.broadcasted_iota(jnp.int32, (BLK, G), 1)
              ).astype(jnp.float32)
    part_sums = jnp.sum(onehot * o, axis=0)[None, :]
    part_cnt = jnp.sum(onehot, axis=0)[None, :]

    @pl.when(i == 0)
    def _():
        sums_ref[...] = jnp.zeros_like(sums_ref)
        cnt_ref[...] = jnp.zeros_like(cnt_ref)

    sums_ref[...] += part_sums
    cnt_ref[...] += part_cnt

    @pl.when(i == NBLK - 1)
    def _():
        out_ref[...] = (sums_ref[...] / jnp.maximum(cnt_ref[...], 1.0)).T


def _tc_final(dp0, dp1, q0, q1, y, W2, b2, Wm1, bm1, Wm2, bm2, batch):
    return pl.pallas_call(
        _final_body,
        grid=(NBLK,),
        in_specs=[
            pl.BlockSpec((BLK, 1), lambda i: (i, 0)),
            pl.BlockSpec((BLK, 1), lambda i: (i, 0)),
            pl.BlockSpec((BLK, D), lambda i: (i, 0)),
            pl.BlockSpec((BLK, D), lambda i: (i, 0)),
            pl.BlockSpec((BLK, D), lambda i: (i, 0)),
            pl.BlockSpec((D, D), lambda i: (0, 0)),
            pl.BlockSpec((1, D), lambda i: (0, 0)),
            pl.BlockSpec((D, H), lambda i: (0, 0)),
            pl.BlockSpec((1, H), lambda i: (0, 0)),
            pl.BlockSpec((H, 1), lambda i: (0, 0)),
            pl.BlockSpec((1, 1), lambda i: (0, 0)),
            pl.BlockSpec((1, 1, BLK), lambda i: (i, 0, 0)),
        ],
        out_specs=pl.BlockSpec((G, 1), lambda i: (0, 0)),
        out_shape=jax.ShapeDtypeStruct((G, 1), jnp.float32),
        scratch_shapes=[
            pltpu.VMEM((1, G), jnp.float32),
            pltpu.VMEM((1, G), jnp.float32),
        ],
    )(dp0, dp1, q0, q1, y, W2, b2.reshape(1, D), Wm1, bm1.reshape(1, H),
      Wm2, bm2.reshape(1, 1), batch.reshape(NBLK, 1, BLK))


def kernel(x, edge_index, edge_weight, batch, W1, b1, W2, b2, Wm1, bm1, Wm2, bm2):
    src = edge_index[0]
    dst = edge_index[1]
    xp = jnp.pad(x, ((0, NP - N), (0, 0)))
    batchp = jnp.pad(batch, (0, NP - N), constant_values=G)

    dp = _sc_degree(dst, edge_weight)
    dp0 = dp[0].reshape(NP, 1)
    dp1 = dp[1].reshape(NP, 1)
    y1 = _tc_prep(dp0, dp1, xp)
    p = _sc_spmm(y1, src, dst, edge_weight)
    y2 = _tc_mid(dp0, dp1, p[0], p[1], y1, W1, b1)
    q = _sc_spmm(y2, src, dst, edge_weight)
    return _tc_final(dp0, dp1, q[0], q[1], y2, W2, b2, Wm1, bm1, Wm2, bm2, batchp)


# speedup vs baseline: 15.3197x; 1.6964x over previous
import jax
import jax.numpy as jnp
from jax import lax
from jax.experimental import pallas as pl
from jax.experimental.pallas import tpu as pltpu
from jax.experimental.pallas import tpu_sc as plsc

N = 10000
E = 320000
D = 128
H = 64
G = 16
NP = 10240

NUM_CORES = 2
NUM_SUBCORES = 16
NUM_TILES = NUM_CORES * NUM_SUBCORES

EDGES_PER_TILE = E // NUM_TILES
K = 80
NCHUNK = EDGES_PER_TILE // K
ROWS_PER_SUB = NP // NUM_SUBCORES
ZROWS = 32


def _sc_mesh():
    return plsc.VectorSubcoreMesh(core_axis_name="c", subcore_axis_name="s",
                                  num_cores=NUM_CORES, num_subcores=NUM_SUBCORES)


NGRP = NP // 128


def _deg_body(meta_hbm, ew_hbm, out_hbm, mv, ewv, dstv, rows, deg_sh, ssems):
    cid = lax.axis_index("c")
    sid = lax.axis_index("s")
    tile_chunk0 = (cid * NUM_SUBCORES + sid) * NCHUNK
    iotas = [lax.iota(jnp.int32, 16) + 16 * j for j in range(8)]

    for r in range(8):
        for j in range(D // 16):
            rows[0][r, pl.ds(j * 16, 16)] = jnp.zeros((16,), jnp.float32)

    @pl.when(sid < NGRP // 8)
    def _():
        pltpu.sync_copy(rows[0].at[pl.ds(0, 8)], deg_sh.at[pl.ds(sid * 8, 8)])

    plsc.subcore_barrier()

    def _stage_meta(ci, b):
        pltpu.sync_copy(meta_hbm.at[pl.ds((tile_chunk0 + ci) * MW, MW)], mv[b])
        pltpu.sync_copy(ew_hbm.at[pl.ds((tile_chunk0 + ci) * K, K)], ewv[b])

    def _scatter_start(b):
        pltpu.async_copy(rows[b], deg_sh.at[dstv[b]], ssems[b], add=True)

    def _scatter_wait(b):
        pltpu.make_async_copy(rows[b], deg_sh.at[dstv[b]], ssems[b]).wait()

    def _compute_chunk(b):
        @pl.loop(0, K // 16)
        def _grp(g):
            sl = pl.ds(g * 16, 16)
            d16 = mv[b][pl.ds(K + g * 16, 16)]
            dstv[b][sl] = lax.shift_right_logical(d16, 7)
            ew16 = ewv[b][sl]
            lane16 = lax.bitwise_and(d16, 127)
            for e in range(16):
                lane = jnp.broadcast_to(lane16[e:e + 1], (16,))
                w = jnp.broadcast_to(ew16[e:e + 1], (16,))
                row = g * 16 + e
                for j in range(8):
                    rows[b][row, pl.ds(j * 16, 16)] = jnp.where(
                        iotas[j] == lane, w, 0.0)

    _stage_meta(0, 0)
    _stage_meta(1, 1)

    @pl.loop(0, NRING // NBUF)
    def _ring(t):
        for b in range(NBUF):
            ci = t * NBUF + b
            _compute_chunk(b)
            _scatter_start(b)
            b2 = (b + 2) % NBUF
            if b < 2:
                @pl.when(t > 0)
                def _():
                    _scatter_wait(b2)
                _stage_meta(ci + 2, b2)
            else:
                @pl.when(t < NRING // NBUF - 1)
                def _():
                    _scatter_wait(b2)
                    _stage_meta(ci + 2, b2)

                @pl.when(t == NRING // NBUF - 1)
                def _():
                    _scatter_wait(b2)

    _stage_meta(NCHUNK - 1, 0)
    _compute_chunk(0)
    _scatter_start(0)
    _scatter_wait(0)
    _scatter_wait(2)
    _scatter_wait(3)

    plsc.subcore_barrier()

    @pl.when(sid < NGRP // 8)
    def _():
        pltpu.sync_copy(deg_sh.at[pl.ds(sid * 8, 8)],
                        out_hbm.at[cid, pl.ds(sid * 8, 8)])


def _sc_degree(meta, ew):
    return pl.kernel(
        _deg_body,
        out_type=jax.ShapeDtypeStruct((NUM_CORES, NGRP, 128), jnp.float32),
        mesh=_sc_mesh(),
        scratch_types=[
            [pltpu.VMEM((MW,), jnp.int32) for _ in range(NBUF)],
            [pltpu.VMEM((K,), jnp.float32) for _ in range(NBUF)],
            [pltpu.VMEM((K,), jnp.int32) for _ in range(NBUF)],
            [pltpu.VMEM((K, 128), jnp.float32) for _ in range(NBUF)],
            pltpu.VMEM_SHARED((NGRP, 128), jnp.float32),
            [pltpu.SemaphoreType.DMA for _ in range(NBUF)],
        ],
    )(meta, ew)


NBUF = 4
NRING = 124
MW = 2 * K


def _spmm_body(y_hbm, meta_hbm, ew_hbm, out_hbm,
               mv, ewv, dstv, rows, acc_sh, gsems, ssems):
    cid = lax.axis_index("c")
    sid = lax.axis_index("s")
    tile_chunk0 = (cid * NUM_SUBCORES + sid) * NCHUNK

    @pl.loop(0, K)
    def _zf(r):
        for j in range(D // 16):
            rows[0][r, pl.ds(j * 16, 16)] = jnp.zeros((16,), jnp.float32)

    row0 = sid * ROWS_PER_SUB

    @pl.loop(0, ROWS_PER_SUB // K)
    def _zero(i):
        pltpu.sync_copy(rows[0], acc_sh.at[pl.ds(row0 + i * K, K)])

    plsc.subcore_barrier()

    def _stage_meta(ci, b):
        pltpu.sync_copy(meta_hbm.at[pl.ds((tile_chunk0 + ci) * MW, MW)], mv[b])
        pltpu.sync_copy(ew_hbm.at[pl.ds((tile_chunk0 + ci) * K, K)], ewv[b])

    def _gather_start(ci, b):
        pltpu.async_copy(y_hbm.at[mv[b].at[pl.ds(0, K)]], rows[b], gsems[b])

    def _gather_wait(ci, b):
        pltpu.make_async_copy(y_hbm.at[mv[b].at[pl.ds(0, K)]], rows[b],
                              gsems[b]).wait()

    def _scatter_start(b):
        pltpu.async_copy(rows[b], acc_sh.at[dstv[b]], ssems[b], add=True)

    def _scatter_wait(b):
        pltpu.make_async_copy(rows[b], acc_sh.at[dstv[b]], ssems[b]).wait()

    def _compute_chunk(b):
        for g in range(K // 16):
            dstv[b][pl.ds(g * 16, 16)] = mv[b][pl.ds(K + g * 16, 16)]

        @pl.loop(0, K // 16)
        def _scale(g):
            ew16 = ewv[b][pl.ds(g * 16, 16)]
            for e in range(16):
                w = jnp.broadcast_to(ew16[e:e + 1], (16,))
                row = g * 16 + e
                for j in range(D // 16):
                    sl = pl.ds(j * 16, 16)
                    rows[b][row, sl] = rows[b][row, sl] * w

    _stage_meta(0, 0)
    _stage_meta(1, 1)
    _gather_start(0, 0)
    _gather_start(1, 1)

    @pl.loop(0, NRING // NBUF)
    def _ring(t):
        for b in range(NBUF):
            ci = t * NBUF + b
            _gather_wait(ci, b)
            _compute_chunk(b)
            _scatter_start(b)
            b2 = (b + 2) % NBUF
            if b < 2:
                @pl.when(t > 0)
                def _():
                    _scatter_wait(b2)
                _stage_meta(ci + 2, b2)
                _gather_start(ci + 2, b2)
            else:
                @pl.when(t < NRING // NBUF - 1)
                def _():
                    _scatter_wait(b2)
                    _stage_meta(ci + 2, b2)
                    _gather_start(ci + 2, b2)

                @pl.when(t == NRING // NBUF - 1)
                def _():
                    _scatter_wait(b2)

    _stage_meta(NCHUNK - 1, 0)
    _gather_start(NCHUNK - 1, 0)
    _gather_wait(NCHUNK - 1, 0)
    _compute_chunk(0)
    _scatter_start(0)
    _scatter_wait(0)
    _scatter_wait(2)
    _scatter_wait(3)

    plsc.subcore_barrier()
    pltpu.sync_copy(acc_sh.at[pl.ds(row0, ROWS_PER_SUB)],
                    out_hbm.at[cid, pl.ds(row0, ROWS_PER_SUB)])


def _sc_spmm(y, meta, ew):
    return pl.kernel(
        _spmm_body,
        out_type=jax.ShapeDtypeStruct((NUM_CORES, NP, D), jnp.float32),
        mesh=_sc_mesh(),
        scratch_types=[
            [pltpu.VMEM((MW,), jnp.int32) for _ in range(NBUF)],
            [pltpu.VMEM((K,), jnp.float32) for _ in range(NBUF)],
            [pltpu.VMEM((K,), jnp.int32) for _ in range(NBUF)],
            [pltpu.VMEM((K, D), jnp.float32) for _ in range(NBUF)],
            pltpu.VMEM_SHARED((NP, D), jnp.float32),
            [pltpu.SemaphoreType.DMA for _ in range(NBUF)],
            [pltpu.SemaphoreType.DMA for _ in range(NBUF)],
        ],
    )(y, meta, ew)


BLK = 1024
NBLK = NP // BLK


def _dinv_from_parts(dp0, dp1):
    deg = dp0 + dp1 + 1.0
    return jnp.where(deg > 0, lax.rsqrt(deg), 0.0)


def _prep_body(dp0_ref, dp1_ref, x_ref, w_ref, y_ref):
    dinv = _dinv_from_parts(dp0_ref[...], dp1_ref[...])
    hw = jnp.dot(x_ref[...], w_ref[...], preferred_element_type=jnp.float32)
    y_ref[...] = hw * dinv


def _tc_prep(dp0, dp1, x, W):
    return pl.pallas_call(
        _prep_body,
        grid=(NBLK,),
        in_specs=[
            pl.BlockSpec((BLK, 1), lambda i: (i, 0)),
            pl.BlockSpec((BLK, 1), lambda i: (i, 0)),
            pl.BlockSpec((BLK, D), lambda i: (i, 0)),
            pl.BlockSpec((D, D), lambda i: (0, 0)),
        ],
        out_specs=pl.BlockSpec((BLK, D), lambda i: (i, 0)),
        out_shape=jax.ShapeDtypeStruct((NP, D), jnp.float32),
    )(dp0, dp1, x, W)


def _mid_body(dp0_ref, dp1_ref, p0_ref, p1_ref, y_ref, w_ref, b_ref, o_ref):
    dinv = _dinv_from_parts(dp0_ref[...], dp1_ref[...])
    z = (p0_ref[...] + p1_ref[...] + y_ref[...]) * dinv + b_ref[...]
    h = jnp.maximum(z, 0.0)
    hw = jnp.dot(h, w_ref[...], preferred_element_type=jnp.float32)
    o_ref[...] = hw * dinv


def _tc_mid(dp0, dp1, p0, p1, y, W, b):
    return pl.pallas_call(
        _mid_body,
        grid=(NBLK,),
        in_specs=[
            pl.BlockSpec((BLK, 1), lambda i: (i, 0)),
            pl.BlockSpec((BLK, 1), lambda i: (i, 0)),
            pl.BlockSpec((BLK, D), lambda i: (i, 0)),
            pl.BlockSpec((BLK, D), lambda i: (i, 0)),
            pl.BlockSpec((BLK, D), lambda i: (i, 0)),
            pl.BlockSpec((D, D), lambda i: (0, 0)),
            pl.BlockSpec((1, D), lambda i: (0, 0)),
        ],
        out_specs=pl.BlockSpec((BLK, D), lambda i: (i, 0)),
        out_shape=jax.ShapeDtypeStruct((NP, D), jnp.float32),
    )(dp0, dp1, p0, p1, y, W, b.reshape(1, D))


def _final_body(dp0_ref, dp1_ref, q0_ref, q1_ref, y_ref, b2_ref,
                wm1_ref, bm1_ref, wm2_ref, bm2_ref, batch_ref,
                out_ref, sums_ref, cnt_ref):
    i = pl.program_id(0)
    dinv = _dinv_from_parts(dp0_ref[...], dp1_ref[...])
    h2 = (q0_ref[...] + q1_ref[...] + y_ref[...]) * dinv + b2_ref[...]
    t = jnp.maximum(jnp.dot(h2, wm1_ref[...],
                            preferred_element_type=jnp.float32) + bm1_ref[...], 0.0)
    o = jnp.dot(t, wm2_ref[...],
                preferred_element_type=jnp.float32) + bm2_ref[...]
    b = batch_ref[...][0]
    onehot = (b[0][:, None] == lax.broadcasted_iota(jnp.int32, (BLK, G), 1)
              ).astype(jnp.float32)
    part_sums = jnp.sum(onehot * o, axis=0)[None, :]
    part_cnt = jnp.sum(onehot, axis=0)[None, :]

    @pl.when(i == 0)
    def _():
        sums_ref[...] = jnp.zeros_like(sums_ref)
        cnt_ref[...] = jnp.zeros_like(cnt_ref)

    sums_ref[...] += part_sums
    cnt_ref[...] += part_cnt

    @pl.when(i == NBLK - 1)
    def _():
        out_ref[...] = (sums_ref[...] / jnp.maximum(cnt_ref[...], 1.0)).T


def _tc_final(dp0, dp1, q0, q1, y, b2, Wm1, bm1, Wm2, bm2, batch):
    return pl.pallas_call(
        _final_body,
        grid=(NBLK,),
        in_specs=[
            pl.BlockSpec((BLK, 1), lambda i: (i, 0)),
            pl.BlockSpec((BLK, 1), lambda i: (i, 0)),
            pl.BlockSpec((BLK, D), lambda i: (i, 0)),
            pl.BlockSpec((BLK, D), lambda i: (i, 0)),
            pl.BlockSpec((BLK, D), lambda i: (i, 0)),
            pl.BlockSpec((1, D), lambda i: (0, 0)),
            pl.BlockSpec((D, H), lambda i: (0, 0)),
            pl.BlockSpec((1, H), lambda i: (0, 0)),
            pl.BlockSpec((H, 1), lambda i: (0, 0)),
            pl.BlockSpec((1, 1), lambda i: (0, 0)),
            pl.BlockSpec((1, 1, BLK), lambda i: (i, 0, 0)),
        ],
        out_specs=pl.BlockSpec((G, 1), lambda i: (0, 0)),
        out_shape=jax.ShapeDtypeStruct((G, 1), jnp.float32),
        scratch_shapes=[
            pltpu.VMEM((1, G), jnp.float32),
            pltpu.VMEM((1, G), jnp.float32),
        ],
    )(dp0, dp1, q0, q1, y, b2.reshape(1, D), Wm1, bm1.reshape(1, H),
      Wm2, bm2.reshape(1, 1), batch.reshape(NBLK, 1, BLK))


def kernel(x, edge_index, edge_weight, batch, W1, b1, W2, b2, Wm1, bm1, Wm2, bm2):
    src = edge_index[0]
    dst = edge_index[1]
    xp = jnp.pad(x, ((0, NP - N), (0, 0)))
    batchp = jnp.pad(batch, (0, NP - N), constant_values=G)

    srcr = src.reshape(NUM_TILES * NCHUNK, 1, K)
    dstr = dst.reshape(NUM_TILES * NCHUNK, 1, K)
    meta = jnp.concatenate([srcr, dstr], axis=1).reshape(-1)

    dp = _sc_degree(meta, edge_weight)
    dp0 = dp[0].reshape(NP, 1)
    dp1 = dp[1].reshape(NP, 1)
    y1 = _tc_prep(dp0, dp1, xp, W1)
    p = _sc_spmm(y1, meta, edge_weight)
    y2 = _tc_mid(dp0, dp1, p[0], p[1], y1, W2, b1)
    q = _sc_spmm(y2, meta, edge_weight)
    return _tc_final(dp0, dp1, q[0], q[1], y2, b2, Wm1, bm1, Wm2, bm2, batchp)
